# baseline (speedup 1.0000x reference)
import functools
import jax
import jax.numpy as jnp
from jax import lax
from jax.experimental import pallas as pl
from jax.experimental.pallas import tpu as pltpu
from jax.experimental.pallas import tpu_sc as plsc

B, S, H = 2, 4096, 2048
N_IMG, P = 4, 256
NC, NS = 2, 16
NW = NC * NS
RPW = B * S // NW
GRP = NW // (B * N_IMG)
QROWS = P // GRP


CHUNK = 16
NCH = RPW // CHUNK


def _sc_body(in_hbm, med_hbm, starts_hbm, out_hbm, starts_v, sems):
    c = lax.axis_index("c")
    s = lax.axis_index("s")
    w = c * NS + s
    base = w * RPW
    pltpu.sync_copy(starts_hbm, starts_v)
    copies = []
    for j in range(NCH):
        r0 = base + jnp.int32(j * CHUNK)
        copies.append(
            pltpu.async_copy(
                in_hbm.at[pl.ds(r0, CHUNK)],
                out_hbm.at[pl.ds(r0, CHUNK)],
                sems.at[jnp.int32(j)],
            )
        )
    for cp in copies:
        cp.wait()
    plsc.subcore_barrier()
    k = w // GRP
    q = w % GRP
    v = starts_v[...]
    lane = lax.broadcasted_iota(jnp.int32, (16,), 0)
    start_k = jnp.max(jnp.where(lane == k, v, 0))
    dst = start_k + q * QROWS
    src = k * P + q * QROWS
    pltpu.sync_copy(med_hbm.at[pl.ds(src, QROWS)], out_hbm.at[pl.ds(dst, QROWS)])


@functools.partial(jax.jit, static_argnums=())
def _sc_call(in2, med2, g16):
    mesh = plsc.VectorSubcoreMesh(
        core_axis_name="c", subcore_axis_name="s", num_cores=NC, num_subcores=NS
    )
    return pl.kernel(
        _sc_body,
        out_type=jax.ShapeDtypeStruct((B * S, H), jnp.float32),
        mesh=mesh,
        scratch_types=[
            pltpu.VMEM((16,), jnp.int32),
            pltpu.SemaphoreType.DMA((NCH,)),
        ],
        compiler_params=pltpu.CompilerParams(
            use_tc_tiling_on_sc=False, needs_layout_passes=False
        ),
    )(in2, med2, g16)


def kernel(inputs_embeds, media_features, media_start_positions):
    in2 = inputs_embeds.reshape(B * S, H)
    med2 = media_features.reshape(B * N_IMG * P, H)
    g = (
        media_start_positions.astype(jnp.int32)
        + jnp.arange(B, dtype=jnp.int32)[:, None] * S
    ).reshape(-1)
    g16 = jnp.pad(g, (0, 16 - B * N_IMG))
    out = _sc_call(in2, med2, g16)
    return out.reshape(B, S, H)


# speedup vs baseline: 12.3461x; 12.3461x over previous
import functools
import jax
import jax.numpy as jnp
from jax import lax
from jax.experimental import pallas as pl
from jax.experimental.pallas import tpu as pltpu
from jax.experimental.pallas import tpu_sc as plsc

B, S, H = 2, 4096, 2048
N_IMG, P = 4, 256
NC, NS = 2, 16
NW = NC * NS
RPW = B * S // NW
GRP = NW // (B * N_IMG)
QROWS = P // GRP
CHUNK = 16
NCH = RPW // CHUNK
NCH2 = QROWS // CHUNK
NBUF = 2


def _stream_rows(src_hbm, dst_hbm, src0, dst0, nch, buf, gsem, ssem, scat):
    for j in range(nch):
        b = j % NBUF
        bi = jnp.int32(b)
        if scat[b] is not None:
            scat[b].wait()
        r = jnp.int32(j * CHUNK)
        pltpu.async_copy(
            src_hbm.at[pl.ds(src0 + r, CHUNK)], buf.at[bi], gsem.at[bi]
        ).wait()
        scat[b] = pltpu.async_copy(
            buf.at[bi], dst_hbm.at[pl.ds(dst0 + r, CHUNK)], ssem.at[bi]
        )
    return scat


def _sc_body(in_hbm, med_hbm, starts_hbm, out_hbm, starts_v, buf, gsem, ssem):
    c = lax.axis_index("c")
    s = lax.axis_index("s")
    w = c * NS + s
    base = w * RPW
    pltpu.sync_copy(starts_hbm, starts_v)
    scat = [None] * NBUF
    scat = _stream_rows(in_hbm, out_hbm, base, base, NCH, buf, gsem, ssem, scat)
    for cp in scat:
        cp.wait()
    plsc.subcore_barrier()
    k = w // GRP
    q = w % GRP
    v = starts_v[...]
    lane = lax.broadcasted_iota(jnp.int32, (16,), 0)
    start_k = jnp.max(jnp.where(lane == k, v, 0))
    dst = start_k + q * QROWS
    src = k * P + q * QROWS
    scat = [None] * NBUF
    scat = _stream_rows(med_hbm, out_hbm, jnp.int32(src), dst, NCH2, buf, gsem, ssem, scat)
    for cp in scat:
        if cp is not None:
            cp.wait()


@jax.jit
def _sc_call(in2, med2, g16):
    mesh = plsc.VectorSubcoreMesh(
        core_axis_name="c", subcore_axis_name="s", num_cores=NC, num_subcores=NS
    )
    return pl.kernel(
        _sc_body,
        out_type=jax.ShapeDtypeStruct((B * S, H), jnp.float32),
        mesh=mesh,
        scratch_types=[
            pltpu.VMEM((16,), jnp.int32),
            pltpu.VMEM((NBUF, CHUNK, H), jnp.float32),
            pltpu.SemaphoreType.DMA((NBUF,)),
            pltpu.SemaphoreType.DMA((NBUF,)),
        ],
        compiler_params=pltpu.CompilerParams(
            use_tc_tiling_on_sc=False, needs_layout_passes=False
        ),
    )(in2, med2, g16)


def kernel(inputs_embeds, media_features, media_start_positions):
    in2 = inputs_embeds.reshape(B * S, H)
    med2 = media_features.reshape(B * N_IMG * P, H)
    g = (
        media_start_positions.astype(jnp.int32)
        + jnp.arange(B, dtype=jnp.int32)[:, None] * S
    ).reshape(-1)
    g16 = jnp.pad(g, (0, 16 - B * N_IMG))
    out = _sc_call(in2, med2, g16)
    return out.reshape(B, S, H)


# speedup vs baseline: 33.1748x; 2.6871x over previous
import jax
import jax.numpy as jnp
from jax import lax
from jax.experimental import pallas as pl
from jax.experimental.pallas import tpu as pltpu
from jax.experimental.pallas import tpu_sc as plsc

B, S, H = 2, 4096, 2048
N_IMG, P = 4, 256
NC, NS = 2, 16
NW = NC * NS
RPW = B * S // NW
GRP = NW // (B * N_IMG)
QROWS = P // GRP
CHUNK = 16
NCH = RPW // CHUNK
NCH2 = QROWS // CHUNK
NBUF = 2


def _sc_body(in_hbm, med_hbm, starts_hbm, out_hbm, starts_v, buf, gsem, ssem):
    c = lax.axis_index("c")
    s = lax.axis_index("s")
    w = c * NS + s
    base = w * RPW
    pltpu.sync_copy(starts_hbm, starts_v)

    scat = [None] * NBUF
    for j in range(NCH):
        b = j % NBUF
        bi = jnp.int32(b)
        if scat[b] is not None:
            scat[b].wait()
        r = base + jnp.int32(j * CHUNK)
        pltpu.async_copy(in_hbm.at[pl.ds(r, CHUNK)], buf.at[bi], gsem.at[bi]).wait()
        scat[b] = pltpu.async_copy(buf.at[bi], out_hbm.at[pl.ds(r, CHUNK)], ssem.at[bi])
    for cp in scat:
        cp.wait()

    plsc.subcore_barrier()

    k = w // GRP
    q = w % GRP
    v = starts_v[...]
    lane = lax.broadcasted_iota(jnp.int32, (16,), 0)
    start_k = jnp.max(jnp.where(lane == k, v, 0))
    iota16 = lax.broadcasted_iota(jnp.int32, (16,), 0)
    scat = [None] * NBUF
    for j in range(NCH2):
        b = j % NBUF
        bi = jnp.int32(b)
        if scat[b] is not None:
            scat[b].wait()
        src = k * P + q * QROWS + jnp.int32(j * CHUNK)
        pltpu.async_copy(med_hbm.at[pl.ds(src, CHUNK)], buf.at[bi], gsem.at[bi]).wait()
        dst_rows = start_k + q * QROWS + jnp.int32(j * CHUNK) + iota16
        scat[b] = pltpu.async_copy(buf.at[bi], out_hbm.at[dst_rows], ssem.at[bi])
    for cp in scat:
        if cp is not None:
            cp.wait()


@jax.jit
def _sc_call(in2, med2, g16):
    mesh = plsc.VectorSubcoreMesh(
        core_axis_name="c", subcore_axis_name="s", num_cores=NC, num_subcores=NS
    )
    return pl.kernel(
        _sc_body,
        out_type=jax.ShapeDtypeStruct((B * S, H), jnp.float32),
        mesh=mesh,
        scratch_types=[
            pltpu.VMEM((16,), jnp.int32),
            pltpu.VMEM((NBUF, CHUNK, H), jnp.float32),
            pltpu.SemaphoreType.DMA((NBUF,)),
            pltpu.SemaphoreType.DMA((NBUF,)),
        ],
        compiler_params=pltpu.CompilerParams(needs_layout_passes=False),
    )(in2, med2, g16)


def kernel(inputs_embeds, media_features, media_start_positions):
    in2 = inputs_embeds.reshape(B * S, H)
    med2 = media_features.reshape(B * N_IMG * P, H)
    g = (
        media_start_positions.astype(jnp.int32)
        + jnp.arange(B, dtype=jnp.int32)[:, None] * S
    ).reshape(-1)
    g16 = jnp.pad(g, (0, 16 - B * N_IMG))
    out = _sc_call(in2, med2, g16)
    return out.reshape(B, S, H)


# speedup vs baseline: 33.2418x; 1.0020x over previous
import jax
import jax.numpy as jnp
from jax import lax
from jax.experimental import pallas as pl
from jax.experimental.pallas import tpu as pltpu
from jax.experimental.pallas import tpu_sc as plsc

B, S, H = 2, 4096, 2048
N_IMG, P = 4, 256
NC, NS = 2, 16
NW = NC * NS
RPW = B * S // NW
GRP = NW // (B * N_IMG)
QROWS = P // GRP
CHUNK = 16
NCH = RPW // CHUNK
NCH2 = QROWS // CHUNK
NBUF = 3


def _sc_body(in_hbm, med_hbm, starts_hbm, out_hbm, starts_v, buf, gsem, ssem):
    c = lax.axis_index("c")
    s = lax.axis_index("s")
    w = c * NS + s
    base = w * RPW
    pltpu.sync_copy(starts_hbm, starts_v)

    def p1_row(j):
        return base + jnp.int32(j * CHUNK)

    gath = [None] * NBUF
    scat = [None] * NBUF
    gath[0] = pltpu.async_copy(
        in_hbm.at[pl.ds(p1_row(0), CHUNK)], buf.at[jnp.int32(0)], gsem.at[jnp.int32(0)]
    )
    for j in range(NCH):
        b = j % NBUF
        bi = jnp.int32(b)
        if j + 1 < NCH:
            b1 = (j + 1) % NBUF
            b1i = jnp.int32(b1)
            if scat[b1] is not None:
                scat[b1].wait()
            gath[b1] = pltpu.async_copy(
                in_hbm.at[pl.ds(p1_row(j + 1), CHUNK)], buf.at[b1i], gsem.at[b1i]
            )
        gath[b].wait()
        scat[b] = pltpu.async_copy(buf.at[bi], out_hbm.at[pl.ds(p1_row(j), CHUNK)], ssem.at[bi])
    for cp in scat:
        if cp is not None:
            cp.wait()

    plsc.subcore_barrier()

    k = w // GRP
    q = w % GRP
    v = starts_v[...]
    lane = lax.broadcasted_iota(jnp.int32, (16,), 0)
    start_k = jnp.max(jnp.where(lane == k, v, 0))
    iota16 = lax.broadcasted_iota(jnp.int32, (16,), 0)

    def p2_src(j):
        return k * P + q * QROWS + jnp.int32(j * CHUNK)

    gath = [None] * NBUF
    scat = [None] * NBUF
    gath[0] = pltpu.async_copy(
        med_hbm.at[pl.ds(p2_src(0), CHUNK)], buf.at[jnp.int32(0)], gsem.at[jnp.int32(0)]
    )
    for j in range(NCH2):
        b = j % NBUF
        bi = jnp.int32(b)
        if j + 1 < NCH2:
            b1 = (j + 1) % NBUF
            b1i = jnp.int32(b1)
            if scat[b1] is not None:
                scat[b1].wait()
            gath[b1] = pltpu.async_copy(
                med_hbm.at[pl.ds(p2_src(j + 1), CHUNK)], buf.at[b1i], gsem.at[b1i]
            )
        gath[b].wait()
        dst_rows = start_k + q * QROWS + jnp.int32(j * CHUNK) + iota16
        scat[b] = pltpu.async_copy(buf.at[bi], out_hbm.at[dst_rows], ssem.at[bi])
    for cp in scat:
        if cp is not None:
            cp.wait()


@jax.jit
def _sc_call(in2, med2, g16):
    mesh = plsc.VectorSubcoreMesh(
        core_axis_name="c", subcore_axis_name="s", num_cores=NC, num_subcores=NS
    )
    return pl.kernel(
        _sc_body,
        out_type=jax.ShapeDtypeStruct((B * S, H), jnp.float32),
        mesh=mesh,
        scratch_types=[
            pltpu.VMEM((16,), jnp.int32),
            pltpu.VMEM((NBUF, CHUNK, H), jnp.float32),
            pltpu.SemaphoreType.DMA((NBUF,)),
            pltpu.SemaphoreType.DMA((NBUF,)),
        ],
        compiler_params=pltpu.CompilerParams(needs_layout_passes=False),
    )(in2, med2, g16)


def kernel(inputs_embeds, media_features, media_start_positions):
    in2 = inputs_embeds.reshape(B * S, H)
    med2 = media_features.reshape(B * N_IMG * P, H)
    g = (
        media_start_positions.astype(jnp.int32)
        + jnp.arange(B, dtype=jnp.int32)[:, None] * S
    ).reshape(-1)
    g16 = jnp.pad(g, (0, 16 - B * N_IMG))
    out = _sc_call(in2, med2, g16)
    return out.reshape(B, S, H)


# speedup vs baseline: 37.3282x; 1.1229x over previous
import jax
import jax.numpy as jnp
from jax import lax
from jax.experimental import pallas as pl
from jax.experimental.pallas import tpu as pltpu
from jax.experimental.pallas import tpu_sc as plsc

B, S, H = 2, 4096, 2048
N_IMG, P = 4, 256
NC, NS = 2, 16
NW = NC * NS
RPW = B * S // NW
GRP = NW // (B * N_IMG)
CHUNK = 16
NCH = RPW // CHUNK
NBUF = 3


def _sc_body(in_hbm, med_hbm, starts_hbm, out_hbm, starts_v, buf, gsem, ssem):
    c = lax.axis_index("c")
    s = lax.axis_index("s")
    w = c * NS + s
    base = w * RPW
    k = w // GRP
    pltpu.sync_copy(starts_hbm, starts_v)

    v = starts_v[...]
    lane = lax.broadcasted_iota(jnp.int32, (16,), 0)
    start_k = jnp.max(jnp.where(lane == k, v, 0))
    iota16 = lane
    med_base = jnp.int32(k * P)

    def issue_gather(j, bi):
        r0 = base + jnp.int32(j * CHUNK)
        interior = (r0 >= start_k) & (r0 + CHUNK <= start_k + P)

        @pl.when(interior)
        def _():
            med_idx = med_base + (r0 - start_k) + iota16
            pltpu.async_copy(med_hbm.at[med_idx], buf.at[bi], gsem.at[bi])

        @pl.when(jnp.logical_not(interior))
        def _():
            pltpu.async_copy(in_hbm.at[pl.ds(r0, CHUNK)], buf.at[bi], gsem.at[bi])

    def drain_gather(bi):
        pltpu.make_async_copy(
            in_hbm.at[pl.ds(jnp.int32(0), CHUNK)], buf.at[bi], gsem.at[bi]
        ).wait()

    scat = [None] * NBUF
    issue_gather(0, jnp.int32(0))
    for j in range(NCH):
        b = j % NBUF
        bi = jnp.int32(b)
        if j + 1 < NCH:
            b1 = (j + 1) % NBUF
            b1i = jnp.int32(b1)
            if scat[b1] is not None:
                scat[b1].wait()
            issue_gather(j + 1, b1i)
        drain_gather(bi)
        r0 = base + jnp.int32(j * CHUNK)
        scat[b] = pltpu.async_copy(buf.at[bi], out_hbm.at[pl.ds(r0, CHUNK)], ssem.at[bi])
    for cp in scat:
        if cp is not None:
            cp.wait()

    end_k = start_k + P

    @pl.when((start_k >= base) & (start_k < base + RPW))
    def _():
        bi = jnp.int32(0)
        pltpu.async_copy(med_hbm.at[pl.ds(med_base, CHUNK)], buf.at[bi], gsem.at[bi]).wait()
        pltpu.async_copy(buf.at[bi], out_hbm.at[start_k + iota16], ssem.at[bi]).wait()

    @pl.when((end_k - 1 >= base) & (end_k - 1 < base + RPW))
    def _():
        bi = jnp.int32(1)
        pltpu.async_copy(
            med_hbm.at[pl.ds(med_base + P - CHUNK, CHUNK)], buf.at[bi], gsem.at[bi]
        ).wait()
        pltpu.async_copy(
            buf.at[bi], out_hbm.at[end_k - CHUNK + iota16], ssem.at[bi]
        ).wait()


@jax.jit
def _sc_call(in2, med2, g16):
    mesh = plsc.VectorSubcoreMesh(
        core_axis_name="c", subcore_axis_name="s", num_cores=NC, num_subcores=NS
    )
    return pl.kernel(
        _sc_body,
        out_type=jax.ShapeDtypeStruct((B * S, H), jnp.float32),
        mesh=mesh,
        scratch_types=[
            pltpu.VMEM((16,), jnp.int32),
            pltpu.VMEM((NBUF, CHUNK, H), jnp.float32),
            pltpu.SemaphoreType.DMA((NBUF,)),
            pltpu.SemaphoreType.DMA((NBUF,)),
        ],
        compiler_params=pltpu.CompilerParams(needs_layout_passes=False),
    )(in2, med2, g16)


def kernel(inputs_embeds, media_features, media_start_positions):
    in2 = inputs_embeds.reshape(B * S, H)
    med2 = media_features.reshape(B * N_IMG * P, H)
    g = (
        media_start_positions.astype(jnp.int32)
        + jnp.arange(B, dtype=jnp.int32)[:, None] * S
    ).reshape(-1)
    g16 = jnp.pad(g, (0, 16 - B * N_IMG))
    out = _sc_call(in2, med2, g16)
    return out.reshape(B, S, H)


# speedup vs baseline: 37.4315x; 1.0028x over previous
import jax
import jax.numpy as jnp
from jax import lax
from jax.experimental import pallas as pl
from jax.experimental.pallas import tpu as pltpu
from jax.experimental.pallas import tpu_sc as plsc

B, S, H = 2, 4096, 2048
N_IMG, P = 4, 256
NC, NS = 2, 16
NW = NC * NS
RPW = B * S // NW
GRP = NW // (B * N_IMG)
CHUNK = 16
NCH = RPW // CHUNK
NBUF = 3


def _sc_body(in_hbm, med_hbm, starts_hbm, out_hbm, starts_v, buf, gsem, ssem):
    c = lax.axis_index("c")
    s = lax.axis_index("s")
    w = c * NS + s
    base = w * RPW
    k = w // GRP
    pltpu.sync_copy(starts_hbm, starts_v)

    v = starts_v[...]
    lane = lax.broadcasted_iota(jnp.int32, (16,), 0)
    start_k = jnp.max(jnp.where(lane == k, v, 0))
    iota16 = lane
    med_base = jnp.int32(k * P)

    def issue_gather(j, bi):
        r0 = base + jnp.int32(j * CHUNK)
        interior = (r0 >= start_k) & (r0 + CHUNK <= start_k + P)

        @pl.when(interior)
        def _():
            med_idx = med_base + (r0 - start_k) + iota16
            pltpu.async_copy(med_hbm.at[med_idx], buf.at[bi], gsem.at[bi])

        @pl.when(jnp.logical_not(interior))
        def _():
            pltpu.async_copy(in_hbm.at[pl.ds(r0, CHUNK)], buf.at[bi], gsem.at[bi])

    def drain_gather(bi):
        pltpu.make_async_copy(
            in_hbm.at[pl.ds(jnp.int32(0), CHUNK)], buf.at[bi], gsem.at[bi]
        ).wait()

    scat = [None] * NBUF
    issue_gather(0, jnp.int32(0))
    for j in range(NCH):
        b = j % NBUF
        bi = jnp.int32(b)
        if j + 1 < NCH:
            b1 = (j + 1) % NBUF
            b1i = jnp.int32(b1)
            if scat[b1] is not None:
                scat[b1].wait()
            issue_gather(j + 1, b1i)
        drain_gather(bi)
        r0 = base + jnp.int32(j * CHUNK)
        scat[b] = pltpu.async_copy(buf.at[bi], out_hbm.at[pl.ds(r0, CHUNK)], ssem.at[bi])
    end_k = start_k + P
    own_head = (start_k >= base) & (start_k < base + RPW)
    own_tail = (end_k - 1 >= base) & (end_k - 1 < base + RPW)
    head_rows = start_k + iota16
    tail_rows = end_k - CHUNK + iota16

    scat[0].wait()

    @pl.when(own_head)
    def _():
        pltpu.async_copy(
            med_hbm.at[pl.ds(med_base, CHUNK)], buf.at[jnp.int32(0)], gsem.at[jnp.int32(0)]
        )

    scat[1].wait()

    @pl.when(own_tail)
    def _():
        pltpu.async_copy(
            med_hbm.at[pl.ds(med_base + P - CHUNK, CHUNK)],
            buf.at[jnp.int32(1)],
            gsem.at[jnp.int32(1)],
        )

    scat[2].wait()

    @pl.when(own_head)
    def _():
        bi = jnp.int32(0)
        pltpu.make_async_copy(med_hbm.at[pl.ds(med_base, CHUNK)], buf.at[bi], gsem.at[bi]).wait()
        pltpu.async_copy(buf.at[bi], out_hbm.at[head_rows], ssem.at[bi])

    @pl.when(own_tail)
    def _():
        bi = jnp.int32(1)
        pltpu.make_async_copy(med_hbm.at[pl.ds(med_base, CHUNK)], buf.at[bi], gsem.at[bi]).wait()
        pltpu.async_copy(buf.at[bi], out_hbm.at[tail_rows], ssem.at[bi])

    @pl.when(own_head)
    def _():
        bi = jnp.int32(0)
        pltpu.make_async_copy(buf.at[bi], out_hbm.at[head_rows], ssem.at[bi]).wait()

    @pl.when(own_tail)
    def _():
        bi = jnp.int32(1)
        pltpu.make_async_copy(buf.at[bi], out_hbm.at[tail_rows], ssem.at[bi]).wait()


@jax.jit
def _sc_call(in2, med2, g16):
    mesh = plsc.VectorSubcoreMesh(
        core_axis_name="c", subcore_axis_name="s", num_cores=NC, num_subcores=NS
    )
    return pl.kernel(
        _sc_body,
        out_type=jax.ShapeDtypeStruct((B * S, H), jnp.float32),
        mesh=mesh,
        scratch_types=[
            pltpu.VMEM((16,), jnp.int32),
            pltpu.VMEM((NBUF, CHUNK, H), jnp.float32),
            pltpu.SemaphoreType.DMA((NBUF,)),
            pltpu.SemaphoreType.DMA((NBUF,)),
        ],
        compiler_params=pltpu.CompilerParams(needs_layout_passes=False),
    )(in2, med2, g16)


def kernel(inputs_embeds, media_features, media_start_positions):
    in2 = inputs_embeds.reshape(B * S, H)
    med2 = media_features.reshape(B * N_IMG * P, H)
    g = (
        media_start_positions.astype(jnp.int32)
        + jnp.arange(B, dtype=jnp.int32)[:, None] * S
    ).reshape(-1)
    g16 = jnp.pad(g, (0, 16 - B * N_IMG))
    out = _sc_call(in2, med2, g16)
    return out.reshape(B, S, H)
